# baseline (speedup 1.0000x reference)
import dataclasses
import functools

import jax
import jax.numpy as jnp
from jax import lax
from jax.experimental import pallas as pl
from jax.experimental.pallas import tpu as pltpu
from jax.experimental.pallas import tpu_sc as plsc

EMB = 128
EMBW = EMB // 2
N_EDGES = 320000
N_TOTAL = 2 * N_EDGES
HALF_E = N_EDGES // 2
NUM_WORKERS = 32
PAIRS = HALF_E // 16
WP = 40
NWIN = PAIRS // WP
NPAIR = NWIN // 2

HB = 1280


def _sc_gather_mul(ei_pos, ei_neg, xp):
    mesh = plsc.VectorSubcoreMesh(
        core_axis_name="c", subcore_axis_name="s", num_cores=2, num_subcores=16
    )

    cp = pltpu.CompilerParams(use_tc_tiling_on_sc=False)
    if "needs_layout_passes" in pltpu.CompilerParams.__dataclass_fields__:
        cp = dataclasses.replace(cp, needs_layout_passes=False)

    @functools.partial(
        pl.kernel,
        out_type=jax.ShapeDtypeStruct((N_TOTAL // 2, EMB), jnp.int32),
        mesh=mesh,
        compiler_params=cp,
        scratch_types=[
            pltpu.VMEM((2 * PAIRS,), jnp.int32),
            pltpu.VMEM((2 * PAIRS,), jnp.int32),
            pltpu.VMEM((2, 2 * WP, EMBW), jnp.int32),
            pltpu.VMEM((2, 2 * WP, EMBW), jnp.int32),
            pltpu.VMEM((2, WP, EMB), jnp.int32),
            pltpu.SemaphoreType.DMA((2,)),
            pltpu.SemaphoreType.DMA((2,)),
            pltpu.SemaphoreType.DMA((2,)),
        ],
    )
    def gather_mul(ei_pos_hbm, ei_neg_hbm, x_hbm, out_hbm, idx_s, idx_d,
                   rows_a, rows_b, rows_c, sem_a, sem_b, sem_o):
        wid = lax.axis_index("s") * 2 + lax.axis_index("c")
        lane = wid % 16
        base_p = lane * PAIRS

        base_c = lane * 2 * PAIRS

        def run(ei_hbm, out_row0):
            pltpu.sync_copy(ei_hbm.at[pl.ds(base_c, 2 * PAIRS)], idx_s)
            pltpu.sync_copy(ei_hbm.at[pl.ds(N_EDGES + base_c, 2 * PAIRS)],
                            idx_d)

            def start_gather(slot, w):
                off = w * 2 * WP
                pltpu.async_copy(x_hbm.at[idx_s.at[pl.ds(off, 2 * WP)]],
                                 rows_a.at[slot], sem_a.at[slot])
                pltpu.async_copy(x_hbm.at[idx_d.at[pl.ds(off, 2 * WP)]],
                                 rows_b.at[slot], sem_b.at[slot])

            def wait_gather(slot):
                pltpu.make_async_copy(x_hbm.at[idx_s.at[pl.ds(0, 2 * WP)]],
                                      rows_a.at[slot], sem_a.at[slot]).wait()
                pltpu.make_async_copy(x_hbm.at[idx_d.at[pl.ds(0, 2 * WP)]],
                                      rows_b.at[slot], sem_b.at[slot]).wait()

            def wait_out(slot):
                pltpu.make_async_copy(
                    rows_c.at[slot], out_hbm.at[pl.ds(out_row0, WP)],
                    sem_o.at[slot]).wait()

            start_gather(0, 0)
            start_gather(1, 1)

            @pl.loop(0, NPAIR)
            def _(p):
                for slot in (0, 1):
                    w = 2 * p + slot
                    wait_gather(slot)

                    @pl.when(p > 0)
                    def _():
                        wait_out(slot)

                    @pl.loop(0, WP, step=2)
                    def _(k):
                        for dk in (0, 1):
                            for half in (0, 1):
                                e = k + dk + half * WP
                                for c in range(EMBW // 16):
                                    csl = pl.ds(c * 16, 16)
                                    va = plsc.bitcast(rows_a[slot, e, csl],
                                                      jnp.bfloat16)
                                    vb = plsc.bitcast(rows_b[slot, e, csl],
                                                      jnp.bfloat16)
                                    prod = jnp.maximum(va * vb,
                                                       jnp.bfloat16(0))
                                    osl = pl.ds(half * EMBW + c * 16, 16)
                                    rows_c[slot, k + dk, osl] = plsc.bitcast(
                                        prod, jnp.int32)

                    pltpu.async_copy(
                        rows_c.at[slot],
                        out_hbm.at[pl.ds(out_row0 + w * WP, WP)],
                        sem_o.at[slot])

                    @pl.when(p < NPAIR - 1)
                    def _():
                        start_gather(slot, w + 2)

            wait_out(0)
            wait_out(1)

        @pl.when(wid < 16)
        def _():
            run(ei_pos_hbm, base_p)

        @pl.when(wid >= 16)
        def _():
            run(ei_neg_hbm, HALF_E + base_p)

    def arrange(ei):
        halves = ei.reshape(2, 2, 16, NWIN, WP)
        return halves.transpose(0, 2, 3, 1, 4).reshape(-1)

    return gather_mul(arrange(ei_pos), arrange(ei_neg), xp)


def _unpack(w):
    bf = jnp.bfloat16
    lo = lax.bitcast_convert_type(
        jnp.left_shift(w, 16), jnp.float32).astype(bf)
    hi = lax.bitcast_convert_type(w, jnp.float32).astype(bf)
    return jnp.concatenate([lo, hi], axis=1)


def _tc_mlp_kernel(em_p_ref, em_n_ref, w1_ref, b1_ref, w2_ref, b2_ref,
                   we1_ref, be1_ref, we2_ref, be2_ref,
                   aa_ref, ab_ref, lpa_ref, lpb_ref, lna_ref, lnb_ref):
    hdims = (((0,), (1,)), ((), ()))
    bf = jnp.bfloat16
    wp = em_p_ref[...]
    wn = em_n_ref[...]
    ds = [_unpack(wp[:, :EMBW]), _unpack(wp[:, EMBW:]),
          _unpack(wn[:, :EMBW]), _unpack(wn[:, EMBW:])]

    def hidden(d, w_ref, b_ref):
        return jnp.maximum(
            jnp.dot(d, w_ref[...],
                    preferred_element_type=jnp.float32).astype(bf)
            + b_ref[...], bf(0))

    def head(h, w_ref, b_ref):
        return jax.nn.sigmoid(
            lax.dot_general(w_ref[...], h, hdims,
                            preferred_element_type=jnp.float32) + b_ref[...])

    hes = [hidden(d, we1_ref, be1_ref) for d in ds]
    lpa_ref[...] = head(hes[0], we2_ref, be2_ref)
    lpb_ref[...] = head(hes[1], we2_ref, be2_ref)
    lna_ref[...] = head(hes[2], we2_ref, be2_ref)
    lnb_ref[...] = head(hes[3], we2_ref, be2_ref)
    aa_ref[...] = head(hidden(ds[0], w1_ref, b1_ref), w2_ref, b2_ref)
    ab_ref[...] = head(hidden(ds[1], w1_ref, b1_ref), w2_ref, b2_ref)


def _perm(w):
    return jnp.concatenate([w[0::2], w[1::2]], axis=0).astype(jnp.bfloat16)


def _tc_mlp(em, W1, b1, W2, b2, We1, be1, We2, be2):
    full = lambda s: pl.BlockSpec(s, lambda i: (0, 0))
    grid = HALF_E // HB
    narrow = lambda j: pl.BlockSpec((j, HB), lambda i: (0, i))
    outs = pl.pallas_call(
        _tc_mlp_kernel,
        grid=(grid,),
        in_specs=[
            pl.BlockSpec((HB, EMB), lambda i: (i, 0)),
            pl.BlockSpec((HB, EMB), lambda i: (grid + i, 0)),
            full((EMB, EMB)),
            full((1, EMB)),
            full((EMB, 7)),
            full((7, 1)),
            full((EMB, EMB)),
            full((1, EMB)),
            full((EMB, 1)),
            full((1, 1)),
        ],
        out_specs=[narrow(7), narrow(7), narrow(1), narrow(1), narrow(1),
                   narrow(1)],
        out_shape=[
            jax.ShapeDtypeStruct((7, HALF_E), jnp.float32),
            jax.ShapeDtypeStruct((7, HALF_E), jnp.float32),
            jax.ShapeDtypeStruct((1, HALF_E), jnp.float32),
            jax.ShapeDtypeStruct((1, HALF_E), jnp.float32),
            jax.ShapeDtypeStruct((1, HALF_E), jnp.float32),
            jax.ShapeDtypeStruct((1, HALF_E), jnp.float32),
        ],
    )(em, em, _perm(W1), b1.reshape(1, EMB).astype(jnp.bfloat16),
      W2.astype(jnp.bfloat16), b2.reshape(7, 1),
      _perm(We1), be1.reshape(1, EMB).astype(jnp.bfloat16),
      We2.astype(jnp.bfloat16), be2.reshape(1, 1))
    return outs


def kernel(x, edge_index, edge_index_neg, W1, b1, W2, b2, We1, be1, We2, be2):
    xb = x.astype(jnp.bfloat16)
    xp = lax.bitcast_convert_type(
        xb.reshape(x.shape[0], EMBW, 2), jnp.int32)
    em = _sc_gather_mul(edge_index, edge_index_neg, xp)
    aa, ab, lpa, lpb, lna, lnb = _tc_mlp(
        em, W1, b1, W2, b2, We1, be1, We2, be2)
    attr = jnp.concatenate([aa, ab], axis=1).T
    edge_pos = jnp.concatenate([lpa[0], lpb[0]])
    edge_neg = jnp.concatenate([lna[0], lnb[0]])
    return (attr, edge_pos, edge_neg)


# speedup vs baseline: 1.4109x; 1.4109x over previous
import dataclasses
import functools

import jax
import jax.numpy as jnp
from jax import lax
from jax.experimental import pallas as pl
from jax.experimental.pallas import tpu as pltpu
from jax.experimental.pallas import tpu_sc as plsc

EMB = 128
EMBW = EMB // 2
N_EDGES = 320000
N_TOTAL = 2 * N_EDGES
HALF_E = N_EDGES // 2
NUM_WORKERS = 32
PAIRS = HALF_E // 16
WP = 40
NWIN = PAIRS // WP
NPAIR = NWIN // 2

HB = 1280


def _sc_gather_mul(ei_pos, ei_neg, xp):
    mesh = plsc.VectorSubcoreMesh(
        core_axis_name="c", subcore_axis_name="s", num_cores=2, num_subcores=16
    )

    cp = pltpu.CompilerParams(use_tc_tiling_on_sc=False)
    if "needs_layout_passes" in pltpu.CompilerParams.__dataclass_fields__:
        cp = dataclasses.replace(cp, needs_layout_passes=False)

    @functools.partial(
        pl.kernel,
        out_type=jax.ShapeDtypeStruct((N_TOTAL // 2, EMB), jnp.int32),
        mesh=mesh,
        compiler_params=cp,
        scratch_types=[
            pltpu.VMEM((2 * PAIRS,), jnp.int32),
            pltpu.VMEM((2 * PAIRS,), jnp.int32),
            pltpu.VMEM((2, 2 * WP, EMBW), jnp.int32),
            pltpu.VMEM((2, 2 * WP, EMBW), jnp.int32),
            pltpu.VMEM((2, WP, EMB), jnp.int32),
            pltpu.SemaphoreType.DMA((2,)),
            pltpu.SemaphoreType.DMA((2,)),
            pltpu.SemaphoreType.DMA((2,)),
        ],
    )
    def gather_mul(ei_pos_hbm, ei_neg_hbm, x_hbm, out_hbm, idx_s, idx_d,
                   rows_a, rows_b, rows_c, sem_a, sem_b, sem_o):
        wid = lax.axis_index("s") * 2 + lax.axis_index("c")
        lane = wid % 16
        base_p = lane * PAIRS

        base_c = lane * 2 * PAIRS

        def run(ei_hbm, out_row0):
            pltpu.sync_copy(ei_hbm.at[pl.ds(base_c, 2 * PAIRS)], idx_s)
            pltpu.sync_copy(ei_hbm.at[pl.ds(N_EDGES + base_c, 2 * PAIRS)],
                            idx_d)

            def start_gather(slot, w):
                off = w * 2 * WP
                pltpu.async_copy(x_hbm.at[idx_s.at[pl.ds(off, 2 * WP)]],
                                 rows_a.at[slot], sem_a.at[slot])
                pltpu.async_copy(x_hbm.at[idx_d.at[pl.ds(off, 2 * WP)]],
                                 rows_b.at[slot], sem_b.at[slot])

            def wait_gather(slot):
                pltpu.make_async_copy(x_hbm.at[idx_s.at[pl.ds(0, 2 * WP)]],
                                      rows_a.at[slot], sem_a.at[slot]).wait()
                pltpu.make_async_copy(x_hbm.at[idx_d.at[pl.ds(0, 2 * WP)]],
                                      rows_b.at[slot], sem_b.at[slot]).wait()

            def wait_out(slot):
                pltpu.make_async_copy(
                    rows_c.at[slot], out_hbm.at[pl.ds(out_row0, WP)],
                    sem_o.at[slot]).wait()

            start_gather(0, 0)
            start_gather(1, 1)

            @pl.loop(0, NPAIR)
            def _(p):
                for slot in (0, 1):
                    w = 2 * p + slot
                    wait_gather(slot)

                    @pl.when(p > 0)
                    def _():
                        wait_out(slot)

                    @plsc.parallel_loop(0, WP, 1, unroll=2)
                    def _(k):
                        for half in (0, 1):
                            e = k + half * WP
                            for c in range(EMBW // 16):
                                csl = pl.ds(c * 16, 16)
                                va = plsc.bitcast(rows_a[slot, e, csl],
                                                  jnp.bfloat16)
                                vb = plsc.bitcast(rows_b[slot, e, csl],
                                                  jnp.bfloat16)
                                prod = jnp.maximum(va * vb, jnp.bfloat16(0))
                                osl = pl.ds(half * EMBW + c * 16, 16)
                                rows_c[slot, k, osl] = plsc.bitcast(
                                    prod, jnp.int32)

                    pltpu.async_copy(
                        rows_c.at[slot],
                        out_hbm.at[pl.ds(out_row0 + w * WP, WP)],
                        sem_o.at[slot])

                    @pl.when(p < NPAIR - 1)
                    def _():
                        start_gather(slot, w + 2)

            wait_out(0)
            wait_out(1)

        @pl.when(wid < 16)
        def _():
            run(ei_pos_hbm, base_p)

        @pl.when(wid >= 16)
        def _():
            run(ei_neg_hbm, HALF_E + base_p)

    def arrange(ei):
        halves = ei.reshape(2, 2, 16, NWIN, WP)
        return halves.transpose(0, 2, 3, 1, 4).reshape(-1)

    return gather_mul(arrange(ei_pos), arrange(ei_neg), xp)


def _unpack(w):
    bf = jnp.bfloat16
    lo = lax.bitcast_convert_type(
        jnp.left_shift(w, 16), jnp.float32).astype(bf)
    hi = lax.bitcast_convert_type(w, jnp.float32).astype(bf)
    return jnp.concatenate([lo, hi], axis=1)


def _tc_mlp_kernel(em_p_ref, em_n_ref, w1_ref, b1_ref, w2_ref, b2_ref,
                   we1_ref, be1_ref, we2_ref, be2_ref,
                   aa_ref, ab_ref, lpa_ref, lpb_ref, lna_ref, lnb_ref):
    hdims = (((0,), (1,)), ((), ()))
    bf = jnp.bfloat16
    wp = em_p_ref[...]
    wn = em_n_ref[...]
    ds = [_unpack(wp[:, :EMBW]), _unpack(wp[:, EMBW:]),
          _unpack(wn[:, :EMBW]), _unpack(wn[:, EMBW:])]

    def hidden(d, w_ref, b_ref):
        return jnp.maximum(
            jnp.dot(d, w_ref[...],
                    preferred_element_type=jnp.float32).astype(bf)
            + b_ref[...], bf(0))

    def head(h, w_ref, b_ref):
        return jax.nn.sigmoid(
            lax.dot_general(w_ref[...], h, hdims,
                            preferred_element_type=jnp.float32) + b_ref[...])

    hes = [hidden(d, we1_ref, be1_ref) for d in ds]
    lpa_ref[...] = head(hes[0], we2_ref, be2_ref)
    lpb_ref[...] = head(hes[1], we2_ref, be2_ref)
    lna_ref[...] = head(hes[2], we2_ref, be2_ref)
    lnb_ref[...] = head(hes[3], we2_ref, be2_ref)
    aa_ref[...] = head(hidden(ds[0], w1_ref, b1_ref), w2_ref, b2_ref)
    ab_ref[...] = head(hidden(ds[1], w1_ref, b1_ref), w2_ref, b2_ref)


def _perm(w):
    return jnp.concatenate([w[0::2], w[1::2]], axis=0).astype(jnp.bfloat16)


def _tc_mlp(em, W1, b1, W2, b2, We1, be1, We2, be2):
    full = lambda s: pl.BlockSpec(s, lambda i: (0, 0))
    grid = HALF_E // HB
    narrow = lambda j: pl.BlockSpec((j, HB), lambda i: (0, i))
    outs = pl.pallas_call(
        _tc_mlp_kernel,
        grid=(grid,),
        in_specs=[
            pl.BlockSpec((HB, EMB), lambda i: (i, 0)),
            pl.BlockSpec((HB, EMB), lambda i: (grid + i, 0)),
            full((EMB, EMB)),
            full((1, EMB)),
            full((EMB, 7)),
            full((7, 1)),
            full((EMB, EMB)),
            full((1, EMB)),
            full((EMB, 1)),
            full((1, 1)),
        ],
        out_specs=[narrow(7), narrow(7), narrow(1), narrow(1), narrow(1),
                   narrow(1)],
        out_shape=[
            jax.ShapeDtypeStruct((7, HALF_E), jnp.float32),
            jax.ShapeDtypeStruct((7, HALF_E), jnp.float32),
            jax.ShapeDtypeStruct((1, HALF_E), jnp.float32),
            jax.ShapeDtypeStruct((1, HALF_E), jnp.float32),
            jax.ShapeDtypeStruct((1, HALF_E), jnp.float32),
            jax.ShapeDtypeStruct((1, HALF_E), jnp.float32),
        ],
    )(em, em, _perm(W1), b1.reshape(1, EMB).astype(jnp.bfloat16),
      W2.astype(jnp.bfloat16), b2.reshape(7, 1),
      _perm(We1), be1.reshape(1, EMB).astype(jnp.bfloat16),
      We2.astype(jnp.bfloat16), be2.reshape(1, 1))
    return outs


def kernel(x, edge_index, edge_index_neg, W1, b1, W2, b2, We1, be1, We2, be2):
    xb = x.astype(jnp.bfloat16)
    xp = lax.bitcast_convert_type(
        xb.reshape(x.shape[0], EMBW, 2), jnp.int32)
    em = _sc_gather_mul(edge_index, edge_index_neg, xp)
    aa, ab, lpa, lpb, lna, lnb = _tc_mlp(
        em, W1, b1, W2, b2, We1, be1, We2, be2)
    attr = jnp.concatenate([aa, ab], axis=1).T
    edge_pos = jnp.concatenate([lpa[0], lpb[0]])
    edge_neg = jnp.concatenate([lna[0], lnb[0]])
    return (attr, edge_pos, edge_neg)


# speedup vs baseline: 1.7184x; 1.2179x over previous
import dataclasses
import functools

import jax
import jax.numpy as jnp
from jax import lax
from jax.experimental import pallas as pl
from jax.experimental.pallas import tpu as pltpu
from jax.experimental.pallas import tpu_sc as plsc

EMB = 128
EMBW = EMB // 2
N_EDGES = 320000
NUM_WORKERS = 32
PAIRS = N_EDGES // NUM_WORKERS
WP = 40
NWIN = PAIRS // WP
NPAIR = NWIN // 2

HB = 2560


def _sc_gather_mul(ei_pos, ei_neg, xp):
    mesh = plsc.VectorSubcoreMesh(
        core_axis_name="c", subcore_axis_name="s", num_cores=2, num_subcores=16
    )

    cp = pltpu.CompilerParams(use_tc_tiling_on_sc=False)
    if "needs_layout_passes" in pltpu.CompilerParams.__dataclass_fields__:
        cp = dataclasses.replace(cp, needs_layout_passes=False)

    @functools.partial(
        pl.kernel,
        out_type=jax.ShapeDtypeStruct((N_EDGES, EMB), jnp.int32),
        mesh=mesh,
        compiler_params=cp,
        scratch_types=[
            pltpu.VMEM((2 * PAIRS,), jnp.int32),
            pltpu.VMEM((2 * PAIRS,), jnp.int32),
            pltpu.VMEM((2, 2 * WP, EMBW), jnp.int32),
            pltpu.VMEM((2, 2 * WP, EMBW), jnp.int32),
            pltpu.VMEM((2, WP, EMB), jnp.int32),
            pltpu.SemaphoreType.DMA((2,)),
            pltpu.SemaphoreType.DMA((2,)),
            pltpu.SemaphoreType.DMA((2,)),
        ],
    )
    def gather_mul(ei_pos_hbm, ei_neg_hbm, x_hbm, out_hbm, idx_s, idx_d,
                   rows_a, rows_b, rows_c, sem_a, sem_b, sem_o):
        wid = lax.axis_index("s") * 2 + lax.axis_index("c")
        base_p = wid * PAIRS

        pltpu.sync_copy(ei_pos_hbm.at[pl.ds(base_p, PAIRS)],
                        idx_s.at[pl.ds(0, PAIRS)])
        pltpu.sync_copy(ei_neg_hbm.at[pl.ds(base_p, PAIRS)],
                        idx_s.at[pl.ds(PAIRS, PAIRS)])
        pltpu.sync_copy(ei_pos_hbm.at[pl.ds(N_EDGES + base_p, PAIRS)],
                        idx_d.at[pl.ds(0, PAIRS)])
        pltpu.sync_copy(ei_neg_hbm.at[pl.ds(N_EDGES + base_p, PAIRS)],
                        idx_d.at[pl.ds(PAIRS, PAIRS)])

        def start_gather(slot, w):
            off = w * WP
            pltpu.async_copy(x_hbm.at[idx_s.at[pl.ds(off, WP)]],
                             rows_a.at[slot, pl.ds(0, WP)], sem_a.at[slot])
            pltpu.async_copy(x_hbm.at[idx_s.at[pl.ds(PAIRS + off, WP)]],
                             rows_a.at[slot, pl.ds(WP, WP)], sem_a.at[slot])
            pltpu.async_copy(x_hbm.at[idx_d.at[pl.ds(off, WP)]],
                             rows_b.at[slot, pl.ds(0, WP)], sem_b.at[slot])
            pltpu.async_copy(x_hbm.at[idx_d.at[pl.ds(PAIRS + off, WP)]],
                             rows_b.at[slot, pl.ds(WP, WP)], sem_b.at[slot])

        def wait_gather(slot):
            pltpu.make_async_copy(x_hbm.at[idx_s.at[pl.ds(0, 2 * WP)]],
                                  rows_a.at[slot], sem_a.at[slot]).wait()
            pltpu.make_async_copy(x_hbm.at[idx_d.at[pl.ds(0, 2 * WP)]],
                                  rows_b.at[slot], sem_b.at[slot]).wait()

        def wait_out(slot):
            pltpu.make_async_copy(
                rows_c.at[slot], out_hbm.at[pl.ds(base_p, WP)],
                sem_o.at[slot]).wait()

        start_gather(0, 0)
        start_gather(1, 1)

        @pl.loop(0, NPAIR)
        def _(p):
            for slot in (0, 1):
                w = 2 * p + slot
                wait_gather(slot)

                @pl.when(p > 0)
                def _():
                    wait_out(slot)

                @plsc.parallel_loop(0, WP, 1, unroll=2)
                def _(k):
                    for half in (0, 1):
                        e = k + half * WP
                        for c in range(EMBW // 16):
                            csl = pl.ds(c * 16, 16)
                            va = plsc.bitcast(rows_a[slot, e, csl],
                                              jnp.bfloat16)
                            vb = plsc.bitcast(rows_b[slot, e, csl],
                                              jnp.bfloat16)
                            prod = jnp.maximum(va * vb, jnp.bfloat16(0))
                            osl = pl.ds(half * EMBW + c * 16, 16)
                            rows_c[slot, k, osl] = plsc.bitcast(
                                prod, jnp.int32)

                pltpu.async_copy(
                    rows_c.at[slot],
                    out_hbm.at[pl.ds(base_p + w * WP, WP)],
                    sem_o.at[slot])

                @pl.when(p < NPAIR - 1)
                def _():
                    start_gather(slot, w + 2)

        wait_out(0)
        wait_out(1)

    return gather_mul(ei_pos.reshape(-1), ei_neg.reshape(-1), xp)


def _unpack(w):
    bf = jnp.bfloat16
    lo = lax.bitcast_convert_type(
        jnp.left_shift(w, 16), jnp.float32).astype(bf)
    hi = lax.bitcast_convert_type(w, jnp.float32).astype(bf)
    return jnp.concatenate([lo, hi], axis=1)


def _tc_mlp_kernel(em_ref, w1_ref, b1_ref, w2_ref, b2_ref,
                   we1_ref, be1_ref, we2_ref, be2_ref,
                   attr_ref, lpos_ref, lneg_ref):
    hdims = (((0,), (1,)), ((), ()))
    bf = jnp.bfloat16
    w = em_ref[...]
    d_p = _unpack(w[:, :EMBW])
    d_n = _unpack(w[:, EMBW:])

    def hidden(d, w_ref, b_ref):
        return jnp.maximum(
            jnp.dot(d, w_ref[...],
                    preferred_element_type=jnp.float32).astype(bf)
            + b_ref[...], bf(0))

    def head(h, w_ref, b_ref):
        return jax.nn.sigmoid(
            lax.dot_general(w_ref[...], h, hdims,
                            preferred_element_type=jnp.float32) + b_ref[...])

    lpos_ref[...] = head(hidden(d_p, we1_ref, be1_ref), we2_ref, be2_ref)
    lneg_ref[...] = head(hidden(d_n, we1_ref, be1_ref), we2_ref, be2_ref)
    attr_ref[...] = head(hidden(d_p, w1_ref, b1_ref), w2_ref, b2_ref)


def _perm(w):
    return jnp.concatenate([w[0::2], w[1::2]], axis=0).astype(jnp.bfloat16)


def _tc_mlp(em, W1, b1, W2, b2, We1, be1, We2, be2):
    full = lambda s: pl.BlockSpec(s, lambda i: (0, 0))
    grid = N_EDGES // HB
    narrow = lambda j: pl.BlockSpec((j, HB), lambda i: (0, i))
    outs = pl.pallas_call(
        _tc_mlp_kernel,
        grid=(grid,),
        in_specs=[
            pl.BlockSpec((HB, EMB), lambda i: (i, 0)),
            full((EMB, EMB)),
            full((1, EMB)),
            full((EMB, 7)),
            full((7, 1)),
            full((EMB, EMB)),
            full((1, EMB)),
            full((EMB, 1)),
            full((1, 1)),
        ],
        out_specs=[narrow(7), narrow(1), narrow(1)],
        out_shape=[
            jax.ShapeDtypeStruct((7, N_EDGES), jnp.float32),
            jax.ShapeDtypeStruct((1, N_EDGES), jnp.float32),
            jax.ShapeDtypeStruct((1, N_EDGES), jnp.float32),
        ],
    )(em, _perm(W1), b1.reshape(1, EMB).astype(jnp.bfloat16),
      W2.astype(jnp.bfloat16), b2.reshape(7, 1),
      _perm(We1), be1.reshape(1, EMB).astype(jnp.bfloat16),
      We2.astype(jnp.bfloat16), be2.reshape(1, 1))
    return outs


def kernel(x, edge_index, edge_index_neg, W1, b1, W2, b2, We1, be1, We2, be2):
    xb = x.astype(jnp.bfloat16)
    xp = lax.bitcast_convert_type(
        xb.reshape(x.shape[0], EMBW, 2), jnp.int32)
    em = _sc_gather_mul(edge_index, edge_index_neg, xp)
    attr_t, lpos, lneg = _tc_mlp(em, W1, b1, W2, b2, We1, be1, We2, be2)
    return (attr_t.T, lpos[0], lneg[0])
